# baseline (speedup 1.0000x reference)
import functools

import jax
import jax.numpy as jnp
from jax import lax
from jax.experimental import pallas as pl
from jax.experimental.pallas import tpu as pltpu
from jax.experimental.pallas import tpu_sc as plsc

E = 160000
N = 10000
CUT = 16
NBINS = 16
M = 16
D = 64
A = 8
ET = A * A + A
DELTA = 6.0
SIGMA = 0.5

NC, NS, L = 2, 16, 16

P_PAD = 163840
RB = P_PAD // 128
RB_SC = RB // NC
RB_TILE = RB_SC // NS
PAIR_SPLIT = P_PAD // 2
GRID_ROWS_SC = 81920
NODE_SPLIT = PAIR_SPLIT // CUT
AGG_ROWS_SC = 5120

N_PAD = 12288
NRB = N_PAD // 128
NRB_TILE = NRB // (NC * NS)

_f32 = jnp.float32
_i32 = jnp.int32


def _mesh():
    return plsc.VectorSubcoreMesh(
        core_axis_name="c", subcore_axis_name="s", num_cores=NC,
        num_subcores=NS)


_SC_PARAMS = pltpu.CompilerParams(use_tc_tiling_on_sc=False)


def _sigmoid(x):
    return 1.0 / (1.0 + jnp.exp(-x))


def _silu(x):
    return x * _sigmoid(x)


def _softmax_lanes(x):
    m = jnp.max(x, axis=1, keepdims=True)
    e = jnp.exp(x - m)
    return e / jnp.sum(e, axis=1, keepdims=True)


def _onehot(idx_col, width, be):
    i = lax.broadcasted_iota(_i32, (be, width), 1)
    return (i == idx_col).astype(_f32)


def _expand_mat(b, out):
    r = lax.broadcasted_iota(_i32, (b, b * out), 0)
    c = lax.broadcasted_iota(_i32, (b, b * out), 1)
    return (c // out == r).astype(_f32)


def _fold_mat(b, out):
    r = lax.broadcasted_iota(_i32, (b * out, out), 0)
    c = lax.broadcasted_iota(_i32, (b * out, out), 1)
    return (r % out == c).astype(_f32)


def _emb_combine(w, basis, b, out):
    wrep = jnp.dot(w, _expand_mat(b, out), preferred_element_type=_f32)
    prod = basis * wrep
    width = b * out
    while width > out:
        width //= 2
        prod = prod[:, :width] + prod[:, width:2 * width]
    return prod


BE1 = 1600


def _k1_body(edge_r, st_r, tt_r, scale_r, off_r, wi_r, bi_r, wm_r, bm_r,
             wasi_r, wati_r, bai_r, was1_r, wat1_r, ba1_r,
             was2_r, wat2_r, ba2_r, wx_r, bx_r,
             msg_r, drm_r, bin_r, w1_r, w2_r):
    ex = edge_r[:, 0:1]
    ey = edge_r[:, 1:2]
    ez = edge_r[:, 2:3]
    d = jnp.sqrt(ex * ex + ey * ey + ez * ez)
    inv = 1.0 / (d + 1e-12)
    ux, uy, uz = ex * inv, ey * inv, ez * inv

    st = st_r[:, 0:1]
    tt = tt_r[:, 0:1]
    oh_s = _onehot(st, A, BE1)
    oh_t = _onehot(tt, A, BE1)

    p1r = lax.broadcasted_iota(_i32, (A, ET), 0)
    p1c = lax.broadcasted_iota(_i32, (A, ET), 1)
    p_src = (p1c % A == p1r).astype(_f32)
    p_tgt = (p1c // A == p1r).astype(_f32)
    oh_et = (jnp.dot(oh_s, p_src, preferred_element_type=_f32)
             * jnp.dot(oh_t, p_tgt, preferred_element_type=_f32))

    sc = jnp.dot(oh_et, scale_r[...], preferred_element_type=_f32)
    of = jnp.dot(oh_et, off_r[...], preferred_element_type=_f32)
    d2 = d * sc + of

    def wtab(was_r, wat_r, ba_r):
        logits = (jnp.dot(p_src.T, was_r[...], preferred_element_type=_f32)
                  + jnp.dot(p_tgt.T, wat_r[...], preferred_element_type=_f32)
                  + ba_r[...])
        return _softmax_lanes(logits)

    wi = jnp.dot(oh_et, wtab(wasi_r, wati_r, bai_r),
                 preferred_element_type=_f32)
    w1_r[...] = jnp.dot(oh_et, wtab(was1_r, wat1_r, ba1_r),
                        preferred_element_type=_f32)
    w2_r[...] = jnp.dot(oh_et, wtab(was2_r, wat2_r, ba2_r),
                        preferred_element_type=_f32)

    d2w = jnp.dot(d2, jnp.ones((1, D), _f32), preferred_element_type=_f32)
    cent = (lax.broadcasted_iota(_i32, (BE1, D), 1).astype(_f32)
            * (DELTA / (D - 1)))
    diff = d2w - cent
    raw = jnp.exp(diff * diff * (-1.0 / (2.0 * SIGMA * SIGMA)))

    dri = jnp.dot(raw, wi_r[...], preferred_element_type=_f32) + bi_r[...]
    drm_r[...] = jnp.dot(raw, wm_r[...], preferred_element_type=_f32) + bm_r[...]

    basis = _silu(jnp.dot(dri, wx_r[...], preferred_element_type=_f32) + bx_r[...])
    wrep = jnp.dot(wi, _expand_mat(A, M), preferred_element_type=_f32)
    msg_r[...] = jnp.dot(basis * wrep, _fold_mat(A, M),
                         preferred_element_type=_f32)

    cq = 0.7071067811865476
    lat = ((uz <= cq).astype(_i32) + (uz <= 0.0).astype(_i32)
           + (uz <= -cq).astype(_i32))
    lon = (jnp.logical_not((ux < 0.0) & (uy < 0.0)).astype(_i32)
           + (uy >= 0.0).astype(_i32)
           + ((ux <= 0.0) & (uy >= 0.0)).astype(_i32))
    bin_r[...] = lat * 4 + lon


def _k1(edge, st2, tt2, p):
    nblk = E // BE1
    full = lambda shape: pl.BlockSpec(shape, lambda i: (0, 0))
    p1, p2 = p['m1_emb'], p['m2_emb']
    return pl.pallas_call(
        _k1_body,
        grid=(nblk,),
        in_specs=[
            pl.BlockSpec((BE1, 3), lambda i: (i, 0)),
            pl.BlockSpec((BE1, 1), lambda i: (i, 0)),
            pl.BlockSpec((BE1, 1), lambda i: (i, 0)),
            full((ET, 1)), full((ET, 1)),
            full((D, D)), full((1, D)), full((D, D)), full((1, D)),
            full((A, A)), full((A, A)), full((1, A)),
            full((A, A)), full((A, A)), full((1, A)),
            full((A, A)), full((A, A)), full((1, A)),
            full((D, A * M)), full((1, A * M)),
        ],
        out_specs=[
            pl.BlockSpec((BE1, M), lambda i: (i, 0)),
            pl.BlockSpec((BE1, D), lambda i: (i, 0)),
            pl.BlockSpec((BE1, 1), lambda i: (i, 0)),
            pl.BlockSpec((BE1, A), lambda i: (i, 0)),
            pl.BlockSpec((BE1, A), lambda i: (i, 0)),
        ],
        out_shape=[
            jax.ShapeDtypeStruct((E, M), _f32),
            jax.ShapeDtypeStruct((E, D), _f32),
            jax.ShapeDtypeStruct((E, 1), _i32),
            jax.ShapeDtypeStruct((E, A), _f32),
            jax.ShapeDtypeStruct((E, A), _f32),
        ],
    )(edge, st2, tt2,
      p['dist_scale'].reshape(ET, 1), p['dist_offset'].reshape(ET, 1),
      p['init_fc_W'], p['init_fc_b'].reshape(1, D),
      p['msg_fc_W'], p['msg_fc_b'].reshape(1, D),
      p['init_emb']['Wa'][:A], p['init_emb']['Wa'][A:],
      p['init_emb']['ba'].reshape(1, A),
      p1['Wa'][:A], p1['Wa'][A:], p1['ba'].reshape(1, A),
      p2['Wa'][:A], p2['Wa'][A:], p2['ba'].reshape(1, A),
      p['init_emb']['Wx'], p['init_emb']['bx'].reshape(1, A * M))


@functools.lru_cache(maxsize=None)
def _make_k3():
    @functools.partial(
        pl.kernel, mesh=_mesh(), compiler_params=_SC_PARAMS,
        out_type=jax.ShapeDtypeStruct((E, M), _f32),
        scratch_types=[
            pltpu.VMEM((RB_TILE, 128), _i32),
            pltpu.VMEM((RB_TILE, 128), _i32),
            pltpu.VMEM((2048, M), _f32),
            pltpu.VMEM((256, M), _f32),
            pltpu.VMEM_SHARED((GRID_ROWS_SC, M), _f32),
            pltpu.SemaphoreType.DMA,
            pltpu.SemaphoreType.DMA,
            pltpu.SemaphoreType.DMA,
            pltpu.SemaphoreType.DMA,
        ],
    )
    def k3(tei2d, bine, msg, grid_out, idx_all_s, idx_all_d, rows,
           zbuf, spm, semg, sems, semz, semb):
        c = lax.axis_index("c")
        s = lax.axis_index("s")
        rb0 = c * RB_SC + s * RB_TILE
        nchunks = RB_TILE // 8

        def zb(r, _):
            zbuf[r, :] = jnp.zeros((L,), _f32)
            return 0
        lax.fori_loop(0, 256, zb, 0)

        rows_tile = GRID_ROWS_SC // NS
        zdescs = [pltpu.async_copy(
            zbuf, spm.at[pl.ds(s * rows_tile + q * 256, 256)], semz)
            for q in range(rows_tile // 256)]

        pltpu.sync_copy(tei2d.at[pl.ds(rb0, RB_TILE)], idx_all_s)

        for grp in range(RB_TILE // 8):
            bdescs = [pltpu.async_copy(
                bine.at[idx_all_s.at[grp * 8 + j]],
                idx_all_d.at[grp * 8 + j], semb) for j in range(8)]
            for dsc in bdescs:
                dsc.wait()

        def dstrow(r, _):
            for v in range(8):
                ibase = (rb0 + r) * 128 + v * L
                ivec = lax.broadcasted_iota(_i32, (L,), 0) + ibase
                row16 = ivec & jnp.int32(-16)
                sub = jnp.where(ivec >= PAIR_SPLIT, jnp.int32(PAIR_SPLIT),
                                jnp.int32(0))
                bvec = idx_all_d[r, pl.ds(v * L, L)]
                idx_all_d[r, pl.ds(v * L, L)] = row16 - sub + bvec
            return 0
        lax.fori_loop(0, RB_TILE, dstrow, 0)

        def fire_gathers(blk):
            par = blk % 2
            return [pltpu.async_copy(
                msg.at[idx_all_s.at[blk * 8 + j]],
                rows.at[pl.ds(par * 1024 + j * 128, 128)], semg)
                for j in range(8)]

        gdescs = fire_gathers(0)
        for dsc in zdescs:
            dsc.wait()
        plsc.subcore_barrier()

        sdescs_prev = None
        for blk in range(nchunks):
            par = blk % 2
            for dsc in gdescs:
                dsc.wait()
            if blk + 1 < nchunks:
                if sdescs_prev is not None:
                    for dsc in sdescs_prev:
                        dsc.wait()
                    sdescs_prev = None
                next_gdescs = fire_gathers(blk + 1)
            sdescs = [pltpu.async_copy(
                rows.at[pl.ds(par * 1024 + j * 128, 128)],
                spm.at[idx_all_d.at[blk * 8 + j]], sems, add=True)
                for j in range(8)]
            if blk + 1 < nchunks:
                sdescs_prev, gdescs = sdescs, next_gdescs
            else:
                for dsc in sdescs:
                    dsc.wait()
        if sdescs_prev is not None:
            for dsc in sdescs_prev:
                dsc.wait()
        plsc.subcore_barrier()

        @pl.when(c == 0)
        def _():
            base = s * 5120
            pltpu.sync_copy(spm.at[pl.ds(base, 5120)],
                            grid_out.at[pl.ds(base, 5120)])

        @pl.when(c == 1)
        def _():
            base = s * 4880
            pltpu.sync_copy(spm.at[pl.ds(base, 4880)],
                            grid_out.at[pl.ds(PAIR_SPLIT + base, 4880)])

    return k3


BN4 = 2000


def _k4_body(grid_r, w_r, b_r, g_r):
    g_r[...] = jnp.dot(grid_r[...], w_r[...], preferred_element_type=_f32) + b_r[...]


def _k4(grid, w, b):
    return pl.pallas_call(
        _k4_body,
        grid=(N // BN4,),
        in_specs=[
            pl.BlockSpec((BN4, NBINS * M), lambda i: (i, 0)),
            pl.BlockSpec((NBINS * M, D), lambda i: (0, 0)),
            pl.BlockSpec((1, D), lambda i: (0, 0)),
        ],
        out_specs=pl.BlockSpec((BN4, D), lambda i: (i, 0)),
        out_shape=jax.ShapeDtypeStruct((N, D), _f32),
    )(grid, w, b.reshape(1, D))


@functools.lru_cache(maxsize=None)
def _make_k5():
    @functools.partial(
        pl.kernel, mesh=_mesh(), compiler_params=_SC_PARAMS,
        out_type=jax.ShapeDtypeStruct((P_PAD, D), _f32),
        scratch_types=[
            pltpu.VMEM((RB_TILE, 128), _i32),
            pltpu.VMEM((1024, D), _f32),
            pltpu.SemaphoreType.DMA,
            pltpu.SemaphoreType.DMA,
        ],
    )
    def k5(esi2d, g, x_out, idx_all, rows, semg, semw):
        c = lax.axis_index("c")
        s = lax.axis_index("s")
        rb0 = c * RB_SC + s * RB_TILE
        nchunks = RB_TILE // 4

        pltpu.sync_copy(esi2d.at[pl.ds(rb0, RB_TILE)], idx_all)

        def fire_gathers(blk):
            par = blk % 2
            return [pltpu.async_copy(
                g.at[idx_all.at[blk * 4 + j]],
                rows.at[pl.ds(par * 512 + j * 128, 128)], semg)
                for j in range(4)]

        gdescs = fire_gathers(0)
        wdesc_prev = None
        for blk in range(nchunks):
            par = blk % 2
            for dsc in gdescs:
                dsc.wait()
            if blk + 1 < nchunks:
                if wdesc_prev is not None:
                    wdesc_prev.wait()
                    wdesc_prev = None
                next_gdescs = fire_gathers(blk + 1)
            wdesc = pltpu.async_copy(
                rows.at[pl.ds(par * 512, 512)],
                x_out.at[pl.ds((rb0 + blk * 4) * 128, 512)], semw)
            if blk + 1 < nchunks:
                wdesc_prev, gdescs = wdesc, next_gdescs
            else:
                wdesc.wait()
        if wdesc_prev is not None:
            wdesc_prev.wait()

    return k5


BE6 = 1600


def _k6_body(x_r, drm_r, msg_r, w1_r, w2_r,
             wx1_r, bx1_r, wx2s_r, bx2_r,
             out_r):
    w1 = w1_r[...]
    w2 = w2_r[...]
    b1 = _silu(jnp.dot(x_r[...], wx1_r[...], preferred_element_type=_f32)
               + bx1_r[...])
    wrep1 = jnp.dot(w1, _expand_mat(A, D), preferred_element_type=_f32)
    prod1 = b1 * wrep1
    f1 = prod1[:, :256] + prod1[:, 256:]
    f2 = f1[:, :128] + f1[:, 128:]
    drm_pad = jnp.concatenate(
        [drm_r[...], jnp.zeros((BE6, D), _f32)], axis=1)
    t2 = f2 + drm_pad
    b2 = _silu(jnp.dot(t2, wx2s_r[...], preferred_element_type=_f32)
               + bx2_r[...])
    wrep2 = jnp.dot(w2, _expand_mat(A, M), preferred_element_type=_f32)
    res = jnp.dot(b2 * wrep2, _fold_mat(A, M), preferred_element_type=_f32)
    out_r[...] = msg_r[...] + res


def _k6(x, drm, msg, w1, w2, p1, p2):
    nblk = E // BE6
    full = lambda shape: pl.BlockSpec(shape, lambda i: (0, 0))
    wx2_stacked = jnp.concatenate([p2['Wx'], p2['Wx']], axis=0)
    return pl.pallas_call(
        _k6_body,
        grid=(nblk,),
        in_specs=[
            pl.BlockSpec((BE6, D), lambda i: (i, 0)),
            pl.BlockSpec((BE6, D), lambda i: (i, 0)),
            pl.BlockSpec((BE6, M), lambda i: (i, 0)),
            pl.BlockSpec((BE6, A), lambda i: (i, 0)),
            pl.BlockSpec((BE6, A), lambda i: (i, 0)),
            full((D, A * D)), full((1, A * D)),
            full((2 * D, A * M)), full((1, A * M)),
        ],
        out_specs=pl.BlockSpec((BE6, M), lambda i: (i, 0)),
        out_shape=jax.ShapeDtypeStruct((E, M), _f32),
    )(x, drm, msg, w1, w2,
      p1['Wx'], p1['bx'].reshape(1, A * D),
      wx2_stacked, p2['bx'].reshape(1, A * M))


@functools.lru_cache(maxsize=None)
def _make_k7():
    @functools.partial(
        pl.kernel, mesh=_mesh(), compiler_params=_SC_PARAMS,
        out_type=[jax.ShapeDtypeStruct((N, M), _f32),
                  jax.ShapeDtypeStruct((NRB, 128), _i32)],
        scratch_types=[
            pltpu.VMEM((RB_TILE, 128), _i32),
            pltpu.VMEM((RB_TILE, 128), _i32),
            pltpu.VMEM((2048, M), _f32),
            pltpu.VMEM((320, M), _f32),
            pltpu.VMEM((NRB_TILE, 128), _i32),
            pltpu.VMEM((NRB_TILE, 128), _i32),
            pltpu.VMEM_SHARED((AGG_ROWS_SC, M), _f32),
            pltpu.SemaphoreType.DMA,
            pltpu.SemaphoreType.DMA,
            pltpu.SemaphoreType.DMA,
            pltpu.SemaphoreType.DMA,
        ],
    )
    def k7(tei2d, msg, tei0, ttv, aggr_out, nt_out,
           idx_all_s, idx_all_d, rows, zbuf, idxn, ntv, spm,
           semg, sems, semz, semn):
        c = lax.axis_index("c")
        s = lax.axis_index("s")
        rb0 = c * RB_SC + s * RB_TILE
        nchunks = RB_TILE // 8

        nb = (c * NS + s) * NRB_TILE
        pltpu.sync_copy(tei0.at[pl.ds(nb, NRB_TILE)], idxn)
        ndescs = [pltpu.async_copy(ttv.at[idxn.at[j]], ntv.at[j], semn)
                  for j in range(NRB_TILE)]

        def zb(r, _):
            zbuf[r, :] = jnp.zeros((L,), _f32)
            return 0
        lax.fori_loop(0, 320, zb, 0)
        zdesc = pltpu.async_copy(zbuf, spm.at[pl.ds(s * 320, 320)], semz)

        pltpu.sync_copy(tei2d.at[pl.ds(rb0, RB_TILE)], idx_all_s)

        def dstrow(r, _):
            for v in range(8):
                ibase = (rb0 + r) * 128 + v * L
                ivec = lax.broadcasted_iota(_i32, (L,), 0) + ibase
                node = lax.shift_right_logical(ivec, 4)
                sub = jnp.where(ivec >= PAIR_SPLIT, jnp.int32(NODE_SPLIT),
                                jnp.int32(0))
                idx_all_d[r, pl.ds(v * L, L)] = node - sub
            return 0
        lax.fori_loop(0, RB_TILE, dstrow, 0)

        def fire_gathers(blk):
            par = blk % 2
            return [pltpu.async_copy(
                msg.at[idx_all_s.at[blk * 8 + j]],
                rows.at[pl.ds(par * 1024 + j * 128, 128)], semg)
                for j in range(8)]

        gdescs = fire_gathers(0)
        for dsc in ndescs:
            dsc.wait()
        pltpu.sync_copy(ntv, nt_out.at[pl.ds(nb, NRB_TILE)])
        zdesc.wait()
        plsc.subcore_barrier()

        sdescs_prev = None
        for blk in range(nchunks):
            par = blk % 2
            for dsc in gdescs:
                dsc.wait()
            if blk + 1 < nchunks:
                if sdescs_prev is not None:
                    for dsc in sdescs_prev:
                        dsc.wait()
                    sdescs_prev = None
                next_gdescs = fire_gathers(blk + 1)
            sdescs = [pltpu.async_copy(
                rows.at[pl.ds(par * 1024 + j * 128, 128)],
                spm.at[idx_all_d.at[blk * 8 + j]], sems, add=True)
                for j in range(8)]
            if blk + 1 < nchunks:
                sdescs_prev, gdescs = sdescs, next_gdescs
            else:
                for dsc in sdescs:
                    dsc.wait()
        if sdescs_prev is not None:
            for dsc in sdescs_prev:
                dsc.wait()
        plsc.subcore_barrier()

        @pl.when(c == 0)
        def _():
            base = s * 320
            pltpu.sync_copy(spm.at[pl.ds(base, 320)],
                            aggr_out.at[pl.ds(base, 320)])

        @pl.when(c == 1)
        def _():
            base = s * 305
            pltpu.sync_copy(spm.at[pl.ds(base, 305)],
                            aggr_out.at[pl.ds(NODE_SPLIT + base, 305)])

    return k7


BN8 = 2000


def _k8_body(aggr_r, nt_r, wae_r, bae_r, wxe_r, bxe_r, out_r):
    nt = nt_r[:, 0:1]
    oh = _onehot(nt, A, BN8)
    w = _softmax_lanes(
        jnp.dot(oh, wae_r[...], preferred_element_type=_f32) + bae_r[...])
    basis = _silu(
        jnp.dot(aggr_r[...], wxe_r[...], preferred_element_type=_f32) + bxe_r[...])
    pe = jnp.sum(w * basis)

    @pl.when(pl.program_id(0) == 0)
    def _():
        out_r[...] = jnp.reshape(pe, (1, 1))

    @pl.when(pl.program_id(0) > 0)
    def _():
        out_r[...] = out_r[...] + jnp.reshape(pe, (1, 1))


def _k8(aggr, nt2, p):
    full = lambda shape: pl.BlockSpec(shape, lambda i: (0, 0))
    return pl.pallas_call(
        _k8_body,
        grid=(N // BN8,),
        in_specs=[
            pl.BlockSpec((BN8, M), lambda i: (i, 0)),
            pl.BlockSpec((BN8, 1), lambda i: (i, 0)),
            full((A, A)), full((1, A)),
            full((M, A)), full((1, A)),
        ],
        out_specs=pl.BlockSpec((1, 1), lambda i: (0, 0)),
        out_shape=jax.ShapeDtypeStruct((1, 1), _f32),
    )(aggr, nt2,
      p['Wa'], p['ba'].reshape(1, A),
      p['Wx'], p['bx'].reshape(1, A))


def kernel(target_edge_index, edge_source_index, edge, source_type,
           target_type, params):
    tei = target_edge_index.astype(_i32)
    esi = edge_source_index.astype(_i32)
    st = source_type.astype(_i32)
    tt = target_type.astype(_i32)

    st2 = st.reshape(E, 1)
    tt2 = tt.reshape(E, 1)

    tei_flat = tei.reshape(-1)
    pad = jnp.zeros((P_PAD - N * CUT,), _i32)
    tei2d = jnp.concatenate([tei_flat, pad]).reshape(RB, 128)
    esi_pad = jnp.concatenate([esi, pad]).reshape(RB, 128)
    tei0 = jnp.concatenate(
        [tei[:, 0], jnp.zeros((N_PAD - N,), _i32)]).reshape(NRB, 128)

    msg, drm, bin2, w1, w2 = _k1(edge, st2, tt2, params)
    bine = bin2.reshape(E)

    for _ in range(2):
        grid = _make_k3()(tei2d, bine, msg)
        g = _k4(grid.reshape(N, NBINS * M), params['spin_fc_W'],
                params['spin_fc_b'])
        x = _make_k5()(esi_pad, g)[:E]
        msg = _k6(x, drm, msg, w1, w2, params['m1_emb'], params['m2_emb'])

    aggr, nt2d = _make_k7()(tei2d, msg, tei0, tt)
    nt = nt2d.reshape(N_PAD)[:N].reshape(N, 1)
    energy = _k8(aggr, nt, params['e_emb'])
    return energy.reshape(())


# speedup vs baseline: 1.0018x; 1.0018x over previous
import functools

import jax
import jax.numpy as jnp
from jax import lax
from jax.experimental import pallas as pl
from jax.experimental.pallas import tpu as pltpu
from jax.experimental.pallas import tpu_sc as plsc

E = 160000
N = 10000
CUT = 16
NBINS = 16
M = 16
D = 64
A = 8
ET = A * A + A
DELTA = 6.0
SIGMA = 0.5

NC, NS, L = 2, 16, 16

P_PAD = 163840
RB = P_PAD // 128
RB_SC = RB // NC
RB_TILE = RB_SC // NS
PAIR_SPLIT = P_PAD // 2
GRID_ROWS_SC = 81920
NODE_SPLIT = PAIR_SPLIT // CUT
AGG_ROWS_SC = 5120

N_PAD = 12288
NRB = N_PAD // 128
NRB_TILE = NRB // (NC * NS)

_f32 = jnp.float32
_i32 = jnp.int32


def _mesh():
    return plsc.VectorSubcoreMesh(
        core_axis_name="c", subcore_axis_name="s", num_cores=NC,
        num_subcores=NS)


_SC_PARAMS = pltpu.CompilerParams(use_tc_tiling_on_sc=False,
                                 skip_device_barrier=True)


def _sigmoid(x):
    return 1.0 / (1.0 + jnp.exp(-x))


def _silu(x):
    return x * _sigmoid(x)


def _softmax_lanes(x):
    m = jnp.max(x, axis=1, keepdims=True)
    e = jnp.exp(x - m)
    return e / jnp.sum(e, axis=1, keepdims=True)


def _onehot(idx_col, width, be):
    i = lax.broadcasted_iota(_i32, (be, width), 1)
    return (i == idx_col).astype(_f32)


def _expand_mat(b, out):
    r = lax.broadcasted_iota(_i32, (b, b * out), 0)
    c = lax.broadcasted_iota(_i32, (b, b * out), 1)
    return (c // out == r).astype(_f32)


def _fold_mat(b, out):
    r = lax.broadcasted_iota(_i32, (b * out, out), 0)
    c = lax.broadcasted_iota(_i32, (b * out, out), 1)
    return (r % out == c).astype(_f32)


def _emb_combine(w, basis, b, out):
    wrep = jnp.dot(w, _expand_mat(b, out), preferred_element_type=_f32)
    prod = basis * wrep
    width = b * out
    while width > out:
        width //= 2
        prod = prod[:, :width] + prod[:, width:2 * width]
    return prod


BE1 = 1600


def _k1_body(edge_r, st_r, tt_r, scale_r, off_r, wi_r, bi_r, wm_r, bm_r,
             wasi_r, wati_r, bai_r, was1_r, wat1_r, ba1_r,
             was2_r, wat2_r, ba2_r, wx_r, bx_r,
             msg_r, drm_r, bin_r, w1_r, w2_r):
    ex = edge_r[:, 0:1]
    ey = edge_r[:, 1:2]
    ez = edge_r[:, 2:3]
    d = jnp.sqrt(ex * ex + ey * ey + ez * ez)
    inv = 1.0 / (d + 1e-12)
    ux, uy, uz = ex * inv, ey * inv, ez * inv

    st = st_r[:, 0:1]
    tt = tt_r[:, 0:1]
    oh_s = _onehot(st, A, BE1)
    oh_t = _onehot(tt, A, BE1)

    p1r = lax.broadcasted_iota(_i32, (A, ET), 0)
    p1c = lax.broadcasted_iota(_i32, (A, ET), 1)
    p_src = (p1c % A == p1r).astype(_f32)
    p_tgt = (p1c // A == p1r).astype(_f32)
    oh_et = (jnp.dot(oh_s, p_src, preferred_element_type=_f32)
             * jnp.dot(oh_t, p_tgt, preferred_element_type=_f32))

    sc = jnp.dot(oh_et, scale_r[...], preferred_element_type=_f32)
    of = jnp.dot(oh_et, off_r[...], preferred_element_type=_f32)
    d2 = d * sc + of

    def wtab(was_r, wat_r, ba_r):
        logits = (jnp.dot(p_src.T, was_r[...], preferred_element_type=_f32)
                  + jnp.dot(p_tgt.T, wat_r[...], preferred_element_type=_f32)
                  + ba_r[...])
        return _softmax_lanes(logits)

    wi = jnp.dot(oh_et, wtab(wasi_r, wati_r, bai_r),
                 preferred_element_type=_f32)
    w1_r[...] = jnp.dot(oh_et, wtab(was1_r, wat1_r, ba1_r),
                        preferred_element_type=_f32)
    w2_r[...] = jnp.dot(oh_et, wtab(was2_r, wat2_r, ba2_r),
                        preferred_element_type=_f32)

    d2w = jnp.dot(d2, jnp.ones((1, D), _f32), preferred_element_type=_f32)
    cent = (lax.broadcasted_iota(_i32, (BE1, D), 1).astype(_f32)
            * (DELTA / (D - 1)))
    diff = d2w - cent
    raw = jnp.exp(diff * diff * (-1.0 / (2.0 * SIGMA * SIGMA)))

    dri = jnp.dot(raw, wi_r[...], preferred_element_type=_f32) + bi_r[...]
    drm_r[...] = jnp.dot(raw, wm_r[...], preferred_element_type=_f32) + bm_r[...]

    basis = _silu(jnp.dot(dri, wx_r[...], preferred_element_type=_f32) + bx_r[...])
    wrep = jnp.dot(wi, _expand_mat(A, M), preferred_element_type=_f32)
    msg_r[...] = jnp.dot(basis * wrep, _fold_mat(A, M),
                         preferred_element_type=_f32)

    cq = 0.7071067811865476
    lat = ((uz <= cq).astype(_i32) + (uz <= 0.0).astype(_i32)
           + (uz <= -cq).astype(_i32))
    lon = (jnp.logical_not((ux < 0.0) & (uy < 0.0)).astype(_i32)
           + (uy >= 0.0).astype(_i32)
           + ((ux <= 0.0) & (uy >= 0.0)).astype(_i32))
    bin_r[...] = lat * 4 + lon


def _k1(edge, st2, tt2, p):
    nblk = E // BE1
    full = lambda shape: pl.BlockSpec(shape, lambda i: (0, 0))
    p1, p2 = p['m1_emb'], p['m2_emb']
    return pl.pallas_call(
        _k1_body,
        grid=(nblk,),
        in_specs=[
            pl.BlockSpec((BE1, 3), lambda i: (i, 0)),
            pl.BlockSpec((BE1, 1), lambda i: (i, 0)),
            pl.BlockSpec((BE1, 1), lambda i: (i, 0)),
            full((ET, 1)), full((ET, 1)),
            full((D, D)), full((1, D)), full((D, D)), full((1, D)),
            full((A, A)), full((A, A)), full((1, A)),
            full((A, A)), full((A, A)), full((1, A)),
            full((A, A)), full((A, A)), full((1, A)),
            full((D, A * M)), full((1, A * M)),
        ],
        out_specs=[
            pl.BlockSpec((BE1, M), lambda i: (i, 0)),
            pl.BlockSpec((BE1, D), lambda i: (i, 0)),
            pl.BlockSpec((BE1, 1), lambda i: (i, 0)),
            pl.BlockSpec((BE1, A), lambda i: (i, 0)),
            pl.BlockSpec((BE1, A), lambda i: (i, 0)),
        ],
        out_shape=[
            jax.ShapeDtypeStruct((E, M), _f32),
            jax.ShapeDtypeStruct((E, D), _f32),
            jax.ShapeDtypeStruct((E, 1), _i32),
            jax.ShapeDtypeStruct((E, A), _f32),
            jax.ShapeDtypeStruct((E, A), _f32),
        ],
    )(edge, st2, tt2,
      p['dist_scale'].reshape(ET, 1), p['dist_offset'].reshape(ET, 1),
      p['init_fc_W'], p['init_fc_b'].reshape(1, D),
      p['msg_fc_W'], p['msg_fc_b'].reshape(1, D),
      p['init_emb']['Wa'][:A], p['init_emb']['Wa'][A:],
      p['init_emb']['ba'].reshape(1, A),
      p1['Wa'][:A], p1['Wa'][A:], p1['ba'].reshape(1, A),
      p2['Wa'][:A], p2['Wa'][A:], p2['ba'].reshape(1, A),
      p['init_emb']['Wx'], p['init_emb']['bx'].reshape(1, A * M))


@functools.lru_cache(maxsize=None)
def _make_k3():
    @functools.partial(
        pl.kernel, mesh=_mesh(), compiler_params=_SC_PARAMS,
        out_type=jax.ShapeDtypeStruct((E, M), _f32),
        scratch_types=[
            pltpu.VMEM((RB_TILE, 128), _i32),
            pltpu.VMEM((RB_TILE, 128), _i32),
            pltpu.VMEM((2048, M), _f32),
            pltpu.VMEM((256, M), _f32),
            pltpu.VMEM_SHARED((GRID_ROWS_SC, M), _f32),
            pltpu.SemaphoreType.DMA,
            pltpu.SemaphoreType.DMA,
            pltpu.SemaphoreType.DMA,
            pltpu.SemaphoreType.DMA,
        ],
    )
    def k3(tei2d, bine, msg, grid_out, idx_all_s, idx_all_d, rows,
           zbuf, spm, semg, sems, semz, semb):
        c = lax.axis_index("c")
        s = lax.axis_index("s")
        rb0 = c * RB_SC + s * RB_TILE
        nchunks = RB_TILE // 8

        def zb(r, _):
            zbuf[r, :] = jnp.zeros((L,), _f32)
            return 0
        lax.fori_loop(0, 256, zb, 0)

        rows_tile = GRID_ROWS_SC // NS
        zdescs = [pltpu.async_copy(
            zbuf, spm.at[pl.ds(s * rows_tile + q * 256, 256)], semz)
            for q in range(rows_tile // 256)]

        pltpu.sync_copy(tei2d.at[pl.ds(rb0, RB_TILE)], idx_all_s)

        for grp in range(RB_TILE // 8):
            bdescs = [pltpu.async_copy(
                bine.at[idx_all_s.at[grp * 8 + j]],
                idx_all_d.at[grp * 8 + j], semb) for j in range(8)]
            for dsc in bdescs:
                dsc.wait()

        def dstrow(r, _):
            for v in range(8):
                ibase = (rb0 + r) * 128 + v * L
                ivec = lax.broadcasted_iota(_i32, (L,), 0) + ibase
                row16 = ivec & jnp.int32(-16)
                sub = jnp.where(ivec >= PAIR_SPLIT, jnp.int32(PAIR_SPLIT),
                                jnp.int32(0))
                bvec = idx_all_d[r, pl.ds(v * L, L)]
                idx_all_d[r, pl.ds(v * L, L)] = row16 - sub + bvec
            return 0
        lax.fori_loop(0, RB_TILE, dstrow, 0)

        def fire_gathers(blk):
            par = blk % 2
            return [pltpu.async_copy(
                msg.at[idx_all_s.at[blk * 8 + j]],
                rows.at[pl.ds(par * 1024 + j * 128, 128)], semg)
                for j in range(8)]

        gdescs = fire_gathers(0)
        for dsc in zdescs:
            dsc.wait()
        plsc.subcore_barrier()

        sdescs_prev = None
        for blk in range(nchunks):
            par = blk % 2
            for dsc in gdescs:
                dsc.wait()
            if blk + 1 < nchunks:
                if sdescs_prev is not None:
                    for dsc in sdescs_prev:
                        dsc.wait()
                    sdescs_prev = None
                next_gdescs = fire_gathers(blk + 1)
            sdescs = [pltpu.async_copy(
                rows.at[pl.ds(par * 1024 + j * 128, 128)],
                spm.at[idx_all_d.at[blk * 8 + j]], sems, add=True)
                for j in range(8)]
            if blk + 1 < nchunks:
                sdescs_prev, gdescs = sdescs, next_gdescs
            else:
                for dsc in sdescs:
                    dsc.wait()
        if sdescs_prev is not None:
            for dsc in sdescs_prev:
                dsc.wait()
        plsc.subcore_barrier()

        @pl.when(c == 0)
        def _():
            base = s * 5120
            pltpu.sync_copy(spm.at[pl.ds(base, 5120)],
                            grid_out.at[pl.ds(base, 5120)])

        @pl.when(c == 1)
        def _():
            base = s * 4880
            pltpu.sync_copy(spm.at[pl.ds(base, 4880)],
                            grid_out.at[pl.ds(PAIR_SPLIT + base, 4880)])

    return k3


BN4 = 2000


def _k4_body(grid_r, w_r, b_r, g_r):
    g_r[...] = jnp.dot(grid_r[...], w_r[...], preferred_element_type=_f32) + b_r[...]


def _k4(grid, w, b):
    return pl.pallas_call(
        _k4_body,
        grid=(N // BN4,),
        in_specs=[
            pl.BlockSpec((BN4, NBINS * M), lambda i: (i, 0)),
            pl.BlockSpec((NBINS * M, D), lambda i: (0, 0)),
            pl.BlockSpec((1, D), lambda i: (0, 0)),
        ],
        out_specs=pl.BlockSpec((BN4, D), lambda i: (i, 0)),
        out_shape=jax.ShapeDtypeStruct((N, D), _f32),
    )(grid, w, b.reshape(1, D))


K5_RB_SC0 = 896
K5_RB_SC1 = RB - K5_RB_SC0


@functools.lru_cache(maxsize=None)
def _make_k5():
    @functools.partial(
        pl.kernel, mesh=_mesh(), compiler_params=_SC_PARAMS,
        out_type=jax.ShapeDtypeStruct((P_PAD, D), _f32),
        scratch_types=[
            pltpu.VMEM((K5_RB_SC0 // NS, 128), _i32),
            pltpu.VMEM((1024, D), _f32),
            pltpu.SemaphoreType.DMA,
            pltpu.SemaphoreType.DMA,
        ],
    )
    def k5(esi2d, g, x_out, idx_all, rows, semg, semw):
        c = lax.axis_index("c")
        s = lax.axis_index("s")

        def pipeline(rb0, rb_tile):
            nchunks = rb_tile // 4
            pltpu.sync_copy(esi2d.at[pl.ds(rb0, rb_tile)],
                            idx_all.at[pl.ds(0, rb_tile)])

            def fire_gathers(blk):
                par = blk % 2
                return [pltpu.async_copy(
                    g.at[idx_all.at[blk * 4 + j]],
                    rows.at[pl.ds(par * 512 + j * 128, 128)], semg)
                    for j in range(4)]

            gdescs = fire_gathers(0)
            wdesc_prev = None
            for blk in range(nchunks):
                par = blk % 2
                for dsc in gdescs:
                    dsc.wait()
                if blk + 1 < nchunks:
                    if wdesc_prev is not None:
                        wdesc_prev.wait()
                        wdesc_prev = None
                    next_gdescs = fire_gathers(blk + 1)
                wdesc = pltpu.async_copy(
                    rows.at[pl.ds(par * 512, 512)],
                    x_out.at[pl.ds((rb0 + blk * 4) * 128, 512)], semw)
                if blk + 1 < nchunks:
                    wdesc_prev, gdescs = wdesc, next_gdescs
                else:
                    wdesc.wait()
            if wdesc_prev is not None:
                wdesc_prev.wait()

        @pl.when(c == 0)
        def _():
            pipeline(s * (K5_RB_SC0 // NS), K5_RB_SC0 // NS)

        @pl.when(c == 1)
        def _():
            pipeline(K5_RB_SC0 + s * (K5_RB_SC1 // NS), K5_RB_SC1 // NS)

    return k5


BE6 = 1600


def _k6_body(x_r, drm_r, msg_r, w1_r, w2_r,
             wx1_r, bx1_r, wx2s_r, bx2_r,
             out_r):
    w1 = w1_r[...]
    w2 = w2_r[...]
    b1 = _silu(jnp.dot(x_r[...], wx1_r[...], preferred_element_type=_f32)
               + bx1_r[...])
    wrep1 = jnp.dot(w1, _expand_mat(A, D), preferred_element_type=_f32)
    prod1 = b1 * wrep1
    f1 = prod1[:, :256] + prod1[:, 256:]
    f2 = f1[:, :128] + f1[:, 128:]
    drm_pad = jnp.concatenate(
        [drm_r[...], jnp.zeros((BE6, D), _f32)], axis=1)
    t2 = f2 + drm_pad
    b2 = _silu(jnp.dot(t2, wx2s_r[...], preferred_element_type=_f32)
               + bx2_r[...])
    wrep2 = jnp.dot(w2, _expand_mat(A, M), preferred_element_type=_f32)
    res = jnp.dot(b2 * wrep2, _fold_mat(A, M), preferred_element_type=_f32)
    out_r[...] = msg_r[...] + res


def _k6(x, drm, msg, w1, w2, p1, p2):
    nblk = E // BE6
    full = lambda shape: pl.BlockSpec(shape, lambda i: (0, 0))
    wx2_stacked = jnp.concatenate([p2['Wx'], p2['Wx']], axis=0)
    return pl.pallas_call(
        _k6_body,
        grid=(nblk,),
        in_specs=[
            pl.BlockSpec((BE6, D), lambda i: (i, 0)),
            pl.BlockSpec((BE6, D), lambda i: (i, 0)),
            pl.BlockSpec((BE6, M), lambda i: (i, 0)),
            pl.BlockSpec((BE6, A), lambda i: (i, 0)),
            pl.BlockSpec((BE6, A), lambda i: (i, 0)),
            full((D, A * D)), full((1, A * D)),
            full((2 * D, A * M)), full((1, A * M)),
        ],
        out_specs=pl.BlockSpec((BE6, M), lambda i: (i, 0)),
        out_shape=jax.ShapeDtypeStruct((E, M), _f32),
    )(x, drm, msg, w1, w2,
      p1['Wx'], p1['bx'].reshape(1, A * D),
      wx2_stacked, p2['bx'].reshape(1, A * M))


@functools.lru_cache(maxsize=None)
def _make_k7():
    @functools.partial(
        pl.kernel, mesh=_mesh(), compiler_params=_SC_PARAMS,
        out_type=[jax.ShapeDtypeStruct((N, M), _f32),
                  jax.ShapeDtypeStruct((NRB, 128), _i32)],
        scratch_types=[
            pltpu.VMEM((RB_TILE, 128), _i32),
            pltpu.VMEM((RB_TILE, 128), _i32),
            pltpu.VMEM((2048, M), _f32),
            pltpu.VMEM((320, M), _f32),
            pltpu.VMEM((NRB_TILE, 128), _i32),
            pltpu.VMEM((NRB_TILE, 128), _i32),
            pltpu.VMEM_SHARED((AGG_ROWS_SC, M), _f32),
            pltpu.SemaphoreType.DMA,
            pltpu.SemaphoreType.DMA,
            pltpu.SemaphoreType.DMA,
            pltpu.SemaphoreType.DMA,
        ],
    )
    def k7(tei2d, msg, tei0, ttv, aggr_out, nt_out,
           idx_all_s, idx_all_d, rows, zbuf, idxn, ntv, spm,
           semg, sems, semz, semn):
        c = lax.axis_index("c")
        s = lax.axis_index("s")
        rb0 = c * RB_SC + s * RB_TILE
        nchunks = RB_TILE // 8

        nb = (c * NS + s) * NRB_TILE
        pltpu.sync_copy(tei0.at[pl.ds(nb, NRB_TILE)], idxn)
        ndescs = [pltpu.async_copy(ttv.at[idxn.at[j]], ntv.at[j], semn)
                  for j in range(NRB_TILE)]

        def zb(r, _):
            zbuf[r, :] = jnp.zeros((L,), _f32)
            return 0
        lax.fori_loop(0, 320, zb, 0)
        zdesc = pltpu.async_copy(zbuf, spm.at[pl.ds(s * 320, 320)], semz)

        pltpu.sync_copy(tei2d.at[pl.ds(rb0, RB_TILE)], idx_all_s)

        def dstrow(r, _):
            for v in range(8):
                ibase = (rb0 + r) * 128 + v * L
                ivec = lax.broadcasted_iota(_i32, (L,), 0) + ibase
                node = lax.shift_right_logical(ivec, 4)
                sub = jnp.where(ivec >= PAIR_SPLIT, jnp.int32(NODE_SPLIT),
                                jnp.int32(0))
                idx_all_d[r, pl.ds(v * L, L)] = node - sub
            return 0
        lax.fori_loop(0, RB_TILE, dstrow, 0)

        def fire_gathers(blk):
            par = blk % 2
            return [pltpu.async_copy(
                msg.at[idx_all_s.at[blk * 8 + j]],
                rows.at[pl.ds(par * 1024 + j * 128, 128)], semg)
                for j in range(8)]

        gdescs = fire_gathers(0)
        for dsc in ndescs:
            dsc.wait()
        pltpu.sync_copy(ntv, nt_out.at[pl.ds(nb, NRB_TILE)])
        zdesc.wait()
        plsc.subcore_barrier()

        sdescs_prev = None
        for blk in range(nchunks):
            par = blk % 2
            for dsc in gdescs:
                dsc.wait()
            if blk + 1 < nchunks:
                if sdescs_prev is not None:
                    for dsc in sdescs_prev:
                        dsc.wait()
                    sdescs_prev = None
                next_gdescs = fire_gathers(blk + 1)
            sdescs = [pltpu.async_copy(
                rows.at[pl.ds(par * 1024 + j * 128, 128)],
                spm.at[idx_all_d.at[blk * 8 + j]], sems, add=True)
                for j in range(8)]
            if blk + 1 < nchunks:
                sdescs_prev, gdescs = sdescs, next_gdescs
            else:
                for dsc in sdescs:
                    dsc.wait()
        if sdescs_prev is not None:
            for dsc in sdescs_prev:
                dsc.wait()
        plsc.subcore_barrier()

        @pl.when(c == 0)
        def _():
            base = s * 320
            pltpu.sync_copy(spm.at[pl.ds(base, 320)],
                            aggr_out.at[pl.ds(base, 320)])

        @pl.when(c == 1)
        def _():
            base = s * 305
            pltpu.sync_copy(spm.at[pl.ds(base, 305)],
                            aggr_out.at[pl.ds(NODE_SPLIT + base, 305)])

    return k7


BN8 = 2000


def _k8_body(aggr_r, nt_r, wae_r, bae_r, wxe_r, bxe_r, out_r):
    nt = nt_r[:, 0:1]
    oh = _onehot(nt, A, BN8)
    w = _softmax_lanes(
        jnp.dot(oh, wae_r[...], preferred_element_type=_f32) + bae_r[...])
    basis = _silu(
        jnp.dot(aggr_r[...], wxe_r[...], preferred_element_type=_f32) + bxe_r[...])
    pe = jnp.sum(w * basis)

    @pl.when(pl.program_id(0) == 0)
    def _():
        out_r[...] = jnp.reshape(pe, (1, 1))

    @pl.when(pl.program_id(0) > 0)
    def _():
        out_r[...] = out_r[...] + jnp.reshape(pe, (1, 1))


def _k8(aggr, nt2, p):
    full = lambda shape: pl.BlockSpec(shape, lambda i: (0, 0))
    return pl.pallas_call(
        _k8_body,
        grid=(N // BN8,),
        in_specs=[
            pl.BlockSpec((BN8, M), lambda i: (i, 0)),
            pl.BlockSpec((BN8, 1), lambda i: (i, 0)),
            full((A, A)), full((1, A)),
            full((M, A)), full((1, A)),
        ],
        out_specs=pl.BlockSpec((1, 1), lambda i: (0, 0)),
        out_shape=jax.ShapeDtypeStruct((1, 1), _f32),
    )(aggr, nt2,
      p['Wa'], p['ba'].reshape(1, A),
      p['Wx'], p['bx'].reshape(1, A))


def kernel(target_edge_index, edge_source_index, edge, source_type,
           target_type, params):
    tei = target_edge_index.astype(_i32)
    esi = edge_source_index.astype(_i32)
    st = source_type.astype(_i32)
    tt = target_type.astype(_i32)

    st2 = st.reshape(E, 1)
    tt2 = tt.reshape(E, 1)

    tei_flat = tei.reshape(-1)
    pad = jnp.zeros((P_PAD - N * CUT,), _i32)
    tei2d = jnp.concatenate([tei_flat, pad]).reshape(RB, 128)
    esi_pad = jnp.concatenate([esi, pad]).reshape(RB, 128)
    tei0 = jnp.concatenate(
        [tei[:, 0], jnp.zeros((N_PAD - N,), _i32)]).reshape(NRB, 128)

    msg, drm, bin2, w1, w2 = _k1(edge, st2, tt2, params)
    bine = bin2.reshape(E)

    for _ in range(2):
        grid = _make_k3()(tei2d, bine, msg)
        g = _k4(grid.reshape(N, NBINS * M), params['spin_fc_W'],
                params['spin_fc_b'])
        x = _make_k5()(esi_pad, g)[:E]
        msg = _k6(x, drm, msg, w1, w2, params['m1_emb'], params['m2_emb'])

    aggr, nt2d = _make_k7()(tei2d, msg, tei0, tt)
    nt = nt2d.reshape(N_PAD)[:N].reshape(N, 1)
    energy = _k8(aggr, nt, params['e_emb'])
    return energy.reshape(())


# speedup vs baseline: 1.0796x; 1.0777x over previous
import functools

import jax
import jax.numpy as jnp
from jax import lax
from jax.experimental import pallas as pl
from jax.experimental.pallas import tpu as pltpu
from jax.experimental.pallas import tpu_sc as plsc

E = 160000
N = 10000
CUT = 16
NBINS = 16
M = 16
D = 64
A = 8
ET = A * A + A
DELTA = 6.0
SIGMA = 0.5

NC, NS, L = 2, 16, 16

P_PAD = 163840
RB = P_PAD // 128
RB_SC = RB // NC
RB_TILE = RB_SC // NS
PAIR_SPLIT = P_PAD // 2
GRID_ROWS_SC = 81920
NODE_SPLIT = PAIR_SPLIT // CUT
AGG_ROWS_SC = 5120

N_PAD = 12288
NRB = N_PAD // 128
NRB_TILE = NRB // (NC * NS)

_f32 = jnp.float32
_i32 = jnp.int32


def _mesh():
    return plsc.VectorSubcoreMesh(
        core_axis_name="c", subcore_axis_name="s", num_cores=NC,
        num_subcores=NS)


_SC_PARAMS = pltpu.CompilerParams(use_tc_tiling_on_sc=False,
                                 skip_device_barrier=True)


def _sigmoid(x):
    return 1.0 / (1.0 + jnp.exp(-x))


def _silu(x):
    return x * _sigmoid(x)


def _softmax_lanes(x):
    m = jnp.max(x, axis=1, keepdims=True)
    e = jnp.exp(x - m)
    return e / jnp.sum(e, axis=1, keepdims=True)


def _onehot(idx_col, width, be):
    i = lax.broadcasted_iota(_i32, (be, width), 1)
    return (i == idx_col).astype(_f32)


def _expand_mat(b, out):
    r = lax.broadcasted_iota(_i32, (b, b * out), 0)
    c = lax.broadcasted_iota(_i32, (b, b * out), 1)
    return (c // out == r).astype(_f32)


def _fold_mat(b, out):
    r = lax.broadcasted_iota(_i32, (b * out, out), 0)
    c = lax.broadcasted_iota(_i32, (b * out, out), 1)
    return (r % out == c).astype(_f32)


def _emb_combine(w, basis, b, out):
    wrep = jnp.dot(w, _expand_mat(b, out), preferred_element_type=_f32)
    prod = basis * wrep
    width = b * out
    while width > out:
        width //= 2
        prod = prod[:, :width] + prod[:, width:2 * width]
    return prod


ER = E // 128


def _k0_body(ex_r, ey_r, ez_r, d_r, bin_r):
    x = ex_r[...]
    y = ey_r[...]
    z = ez_r[...]
    d = jnp.sqrt(x * x + y * y + z * z)
    d_r[...] = d
    cq = 0.7071067811865476
    lat = ((z <= cq * d).astype(_i32) + (z <= 0.0).astype(_i32)
           + (z <= -cq * d).astype(_i32))
    lon = (jnp.logical_not((x < 0.0) & (y < 0.0)).astype(_i32)
           + (y >= 0.0).astype(_i32)
           + ((x <= 0.0) & (y >= 0.0)).astype(_i32))
    bin_r[...] = lat * 4 + lon


def _k0(ex, ey, ez):
    blk = lambda: pl.BlockSpec((ER, 128), lambda i: (0, 0))
    return pl.pallas_call(
        _k0_body,
        grid=(1,),
        in_specs=[blk(), blk(), blk()],
        out_specs=[blk(), blk()],
        out_shape=[
            jax.ShapeDtypeStruct((ER, 128), _f32),
            jax.ShapeDtypeStruct((ER, 128), _i32),
        ],
    )(ex, ey, ez)


BE1 = 1600


def _k1_body(d_r, st_r, tt_r, scale_r, off_r, wi_r, bi_r, wm_r, bm_r,
             wasi_r, wati_r, bai_r, was1_r, wat1_r, ba1_r,
             was2_r, wat2_r, ba2_r, wx_r, bx_r,
             msg_r, drm_r, w1_r, w2_r):
    st = st_r[:, 0:1]
    tt = tt_r[:, 0:1]
    oh_s = _onehot(st, A, BE1)
    oh_t = _onehot(tt, A, BE1)

    p1r = lax.broadcasted_iota(_i32, (A, ET), 0)
    p1c = lax.broadcasted_iota(_i32, (A, ET), 1)
    p_src = (p1c % A == p1r).astype(_f32)
    p_tgt = (p1c // A == p1r).astype(_f32)
    oh_et = (jnp.dot(oh_s, p_src, preferred_element_type=_f32)
             * jnp.dot(oh_t, p_tgt, preferred_element_type=_f32))

    ones64 = jnp.ones((1, D), _f32)
    scw = jnp.dot(oh_et, jnp.dot(scale_r[...], ones64,
                                 preferred_element_type=_f32),
                  preferred_element_type=_f32)
    ofw = jnp.dot(oh_et, jnp.dot(off_r[...], ones64,
                                 preferred_element_type=_f32),
                  preferred_element_type=_f32)

    def wtab(was_r, wat_r, ba_r):
        logits = (jnp.dot(p_src.T, was_r[...], preferred_element_type=_f32)
                  + jnp.dot(p_tgt.T, wat_r[...], preferred_element_type=_f32)
                  + ba_r[...])
        return _softmax_lanes(logits)

    wi = jnp.dot(oh_et, wtab(wasi_r, wati_r, bai_r),
                 preferred_element_type=_f32)
    w1_r[...] = jnp.dot(oh_et, wtab(was1_r, wat1_r, ba1_r),
                        preferred_element_type=_f32)
    w2_r[...] = jnp.dot(oh_et, wtab(was2_r, wat2_r, ba2_r),
                        preferred_element_type=_f32)

    dw = jnp.dot(d_r[...], jnp.ones((1, D), _f32),
                 preferred_element_type=_f32)
    d2w = dw * scw + ofw
    cent = (lax.broadcasted_iota(_i32, (BE1, D), 1).astype(_f32)
            * (DELTA / (D - 1)))
    diff = d2w - cent
    raw = jnp.exp(diff * diff * (-1.0 / (2.0 * SIGMA * SIGMA)))

    dri = jnp.dot(raw, wi_r[...], preferred_element_type=_f32) + bi_r[...]
    drm_r[...] = jnp.dot(raw, wm_r[...], preferred_element_type=_f32) + bm_r[...]

    basis = _silu(jnp.dot(dri, wx_r[...], preferred_element_type=_f32) + bx_r[...])
    wrep = jnp.dot(wi, _expand_mat(A, M), preferred_element_type=_f32)
    msg_r[...] = jnp.dot(basis * wrep, _fold_mat(A, M),
                         preferred_element_type=_f32)


def _k1(d2col, st2, tt2, p):
    nblk = E // BE1
    full = lambda shape: pl.BlockSpec(shape, lambda i: (0, 0))
    p1, p2 = p['m1_emb'], p['m2_emb']
    return pl.pallas_call(
        _k1_body,
        grid=(nblk,),
        in_specs=[
            pl.BlockSpec((BE1, 1), lambda i: (i, 0)),
            pl.BlockSpec((BE1, 1), lambda i: (i, 0)),
            pl.BlockSpec((BE1, 1), lambda i: (i, 0)),
            full((ET, 1)), full((ET, 1)),
            full((D, D)), full((1, D)), full((D, D)), full((1, D)),
            full((A, A)), full((A, A)), full((1, A)),
            full((A, A)), full((A, A)), full((1, A)),
            full((A, A)), full((A, A)), full((1, A)),
            full((D, A * M)), full((1, A * M)),
        ],
        out_specs=[
            pl.BlockSpec((BE1, M), lambda i: (i, 0)),
            pl.BlockSpec((BE1, D), lambda i: (i, 0)),
            pl.BlockSpec((BE1, A), lambda i: (i, 0)),
            pl.BlockSpec((BE1, A), lambda i: (i, 0)),
        ],
        out_shape=[
            jax.ShapeDtypeStruct((E, M), _f32),
            jax.ShapeDtypeStruct((E, D), _f32),
            jax.ShapeDtypeStruct((E, A), _f32),
            jax.ShapeDtypeStruct((E, A), _f32),
        ],
    )(d2col, st2, tt2,
      p['dist_scale'].reshape(ET, 1), p['dist_offset'].reshape(ET, 1),
      p['init_fc_W'], p['init_fc_b'].reshape(1, D),
      p['msg_fc_W'], p['msg_fc_b'].reshape(1, D),
      p['init_emb']['Wa'][:A], p['init_emb']['Wa'][A:],
      p['init_emb']['ba'].reshape(1, A),
      p1['Wa'][:A], p1['Wa'][A:], p1['ba'].reshape(1, A),
      p2['Wa'][:A], p2['Wa'][A:], p2['ba'].reshape(1, A),
      p['init_emb']['Wx'], p['init_emb']['bx'].reshape(1, A * M))


@functools.lru_cache(maxsize=None)
def _make_k3():
    @functools.partial(
        pl.kernel, mesh=_mesh(), compiler_params=_SC_PARAMS,
        out_type=jax.ShapeDtypeStruct((E, M), _f32),
        scratch_types=[
            pltpu.VMEM((RB_TILE, 128), _i32),
            pltpu.VMEM((RB_TILE, 128), _i32),
            pltpu.VMEM((2048, M), _f32),
            pltpu.VMEM((256, M), _f32),
            pltpu.VMEM_SHARED((GRID_ROWS_SC, M), _f32),
            pltpu.SemaphoreType.DMA,
            pltpu.SemaphoreType.DMA,
            pltpu.SemaphoreType.DMA,
            pltpu.SemaphoreType.DMA,
        ],
    )
    def k3(tei2d, bine, msg, grid_out, idx_all_s, idx_all_d, rows,
           zbuf, spm, semg, sems, semz, semb):
        c = lax.axis_index("c")
        s = lax.axis_index("s")
        rb0 = c * RB_SC + s * RB_TILE
        nchunks = RB_TILE // 8

        def zb(r, _):
            zbuf[r, :] = jnp.zeros((L,), _f32)
            return 0
        lax.fori_loop(0, 256, zb, 0)

        rows_tile = GRID_ROWS_SC // NS
        zdescs = [pltpu.async_copy(
            zbuf, spm.at[pl.ds(s * rows_tile + q * 256, 256)], semz)
            for q in range(rows_tile // 256)]

        pltpu.sync_copy(tei2d.at[pl.ds(rb0, RB_TILE)], idx_all_s)

        for grp in range(RB_TILE // 8):
            bdescs = [pltpu.async_copy(
                bine.at[idx_all_s.at[grp * 8 + j]],
                idx_all_d.at[grp * 8 + j], semb) for j in range(8)]
            for dsc in bdescs:
                dsc.wait()

        def dstrow(r, _):
            for v in range(8):
                ibase = (rb0 + r) * 128 + v * L
                ivec = lax.broadcasted_iota(_i32, (L,), 0) + ibase
                row16 = ivec & jnp.int32(-16)
                sub = jnp.where(ivec >= PAIR_SPLIT, jnp.int32(PAIR_SPLIT),
                                jnp.int32(0))
                bvec = idx_all_d[r, pl.ds(v * L, L)]
                idx_all_d[r, pl.ds(v * L, L)] = row16 - sub + bvec
            return 0
        lax.fori_loop(0, RB_TILE, dstrow, 0)

        def fire_gathers(blk):
            par = blk % 2
            return [pltpu.async_copy(
                msg.at[idx_all_s.at[blk * 8 + j]],
                rows.at[pl.ds(par * 1024 + j * 128, 128)], semg)
                for j in range(8)]

        gdescs = fire_gathers(0)
        for dsc in zdescs:
            dsc.wait()
        plsc.subcore_barrier()

        sdescs_prev = None
        for blk in range(nchunks):
            par = blk % 2
            for dsc in gdescs:
                dsc.wait()
            if blk + 1 < nchunks:
                if sdescs_prev is not None:
                    for dsc in sdescs_prev:
                        dsc.wait()
                    sdescs_prev = None
                next_gdescs = fire_gathers(blk + 1)
            sdescs = [pltpu.async_copy(
                rows.at[pl.ds(par * 1024 + j * 128, 128)],
                spm.at[idx_all_d.at[blk * 8 + j]], sems, add=True)
                for j in range(8)]
            if blk + 1 < nchunks:
                sdescs_prev, gdescs = sdescs, next_gdescs
            else:
                for dsc in sdescs:
                    dsc.wait()
        if sdescs_prev is not None:
            for dsc in sdescs_prev:
                dsc.wait()
        plsc.subcore_barrier()

        @pl.when(c == 0)
        def _():
            base = s * 5120
            pltpu.sync_copy(spm.at[pl.ds(base, 5120)],
                            grid_out.at[pl.ds(base, 5120)])

        @pl.when(c == 1)
        def _():
            base = s * 4880
            pltpu.sync_copy(spm.at[pl.ds(base, 4880)],
                            grid_out.at[pl.ds(PAIR_SPLIT + base, 4880)])

    return k3


BN4 = 2000


def _k4_body(grid_r, w_r, b_r, g_r):
    g_r[...] = jnp.dot(grid_r[...], w_r[...], preferred_element_type=_f32) + b_r[...]


def _k4(grid, w, b):
    return pl.pallas_call(
        _k4_body,
        grid=(N // BN4,),
        in_specs=[
            pl.BlockSpec((BN4, NBINS * M), lambda i: (i, 0)),
            pl.BlockSpec((NBINS * M, D), lambda i: (0, 0)),
            pl.BlockSpec((1, D), lambda i: (0, 0)),
        ],
        out_specs=pl.BlockSpec((BN4, D), lambda i: (i, 0)),
        out_shape=jax.ShapeDtypeStruct((N, D), _f32),
    )(grid, w, b.reshape(1, D))


K5_RB_SC0 = 896
K5_RB_SC1 = RB - K5_RB_SC0


@functools.lru_cache(maxsize=None)
def _make_k5():
    @functools.partial(
        pl.kernel, mesh=_mesh(), compiler_params=_SC_PARAMS,
        out_type=jax.ShapeDtypeStruct((P_PAD, D), _f32),
        scratch_types=[
            pltpu.VMEM((K5_RB_SC0 // NS, 128), _i32),
            pltpu.VMEM((1024, D), _f32),
            pltpu.SemaphoreType.DMA,
            pltpu.SemaphoreType.DMA,
        ],
    )
    def k5(esi2d, g, x_out, idx_all, rows, semg, semw):
        c = lax.axis_index("c")
        s = lax.axis_index("s")

        def pipeline(rb0, rb_tile):
            nchunks = rb_tile // 4
            pltpu.sync_copy(esi2d.at[pl.ds(rb0, rb_tile)],
                            idx_all.at[pl.ds(0, rb_tile)])

            def fire_gathers(blk):
                par = blk % 2
                return [pltpu.async_copy(
                    g.at[idx_all.at[blk * 4 + j]],
                    rows.at[pl.ds(par * 512 + j * 128, 128)], semg)
                    for j in range(4)]

            gdescs = fire_gathers(0)
            wdesc_prev = None
            for blk in range(nchunks):
                par = blk % 2
                for dsc in gdescs:
                    dsc.wait()
                if blk + 1 < nchunks:
                    if wdesc_prev is not None:
                        wdesc_prev.wait()
                        wdesc_prev = None
                    next_gdescs = fire_gathers(blk + 1)
                wdesc = pltpu.async_copy(
                    rows.at[pl.ds(par * 512, 512)],
                    x_out.at[pl.ds((rb0 + blk * 4) * 128, 512)], semw)
                if blk + 1 < nchunks:
                    wdesc_prev, gdescs = wdesc, next_gdescs
                else:
                    wdesc.wait()
            if wdesc_prev is not None:
                wdesc_prev.wait()

        @pl.when(c == 0)
        def _():
            pipeline(s * (K5_RB_SC0 // NS), K5_RB_SC0 // NS)

        @pl.when(c == 1)
        def _():
            pipeline(K5_RB_SC0 + s * (K5_RB_SC1 // NS), K5_RB_SC1 // NS)

    return k5


BE6 = 1600


def _k6_body(x_r, drm_r, msg_r, w1_r, w2_r,
             wx1_r, bx1_r, wx2s_r, bx2_r,
             out_r):
    w1 = w1_r[...]
    w2 = w2_r[...]
    b1 = _silu(jnp.dot(x_r[...], wx1_r[...], preferred_element_type=_f32)
               + bx1_r[...])
    wrep1 = jnp.dot(w1, _expand_mat(A, D), preferred_element_type=_f32)
    prod1 = b1 * wrep1
    f1 = prod1[:, :256] + prod1[:, 256:]
    f2 = f1[:, :128] + f1[:, 128:]
    drm_pad = jnp.concatenate(
        [drm_r[...], jnp.zeros((BE6, D), _f32)], axis=1)
    t2 = f2 + drm_pad
    b2 = _silu(jnp.dot(t2, wx2s_r[...], preferred_element_type=_f32)
               + bx2_r[...])
    wrep2 = jnp.dot(w2, _expand_mat(A, M), preferred_element_type=_f32)
    res = jnp.dot(b2 * wrep2, _fold_mat(A, M), preferred_element_type=_f32)
    out_r[...] = msg_r[...] + res


def _k6(x, drm, msg, w1, w2, p1, p2):
    nblk = E // BE6
    full = lambda shape: pl.BlockSpec(shape, lambda i: (0, 0))
    wx2_stacked = jnp.concatenate([p2['Wx'], p2['Wx']], axis=0)
    return pl.pallas_call(
        _k6_body,
        grid=(nblk,),
        in_specs=[
            pl.BlockSpec((BE6, D), lambda i: (i, 0)),
            pl.BlockSpec((BE6, D), lambda i: (i, 0)),
            pl.BlockSpec((BE6, M), lambda i: (i, 0)),
            pl.BlockSpec((BE6, A), lambda i: (i, 0)),
            pl.BlockSpec((BE6, A), lambda i: (i, 0)),
            full((D, A * D)), full((1, A * D)),
            full((2 * D, A * M)), full((1, A * M)),
        ],
        out_specs=pl.BlockSpec((BE6, M), lambda i: (i, 0)),
        out_shape=jax.ShapeDtypeStruct((E, M), _f32),
    )(x, drm, msg, w1, w2,
      p1['Wx'], p1['bx'].reshape(1, A * D),
      wx2_stacked, p2['bx'].reshape(1, A * M))


@functools.lru_cache(maxsize=None)
def _make_k7():
    @functools.partial(
        pl.kernel, mesh=_mesh(), compiler_params=_SC_PARAMS,
        out_type=[jax.ShapeDtypeStruct((N, M), _f32),
                  jax.ShapeDtypeStruct((NRB, 128), _i32)],
        scratch_types=[
            pltpu.VMEM((RB_TILE, 128), _i32),
            pltpu.VMEM((RB_TILE, 128), _i32),
            pltpu.VMEM((2048, M), _f32),
            pltpu.VMEM((320, M), _f32),
            pltpu.VMEM((NRB_TILE, 128), _i32),
            pltpu.VMEM((NRB_TILE, 128), _i32),
            pltpu.VMEM_SHARED((AGG_ROWS_SC, M), _f32),
            pltpu.SemaphoreType.DMA,
            pltpu.SemaphoreType.DMA,
            pltpu.SemaphoreType.DMA,
            pltpu.SemaphoreType.DMA,
        ],
    )
    def k7(tei2d, msg, tei0, ttv, aggr_out, nt_out,
           idx_all_s, idx_all_d, rows, zbuf, idxn, ntv, spm,
           semg, sems, semz, semn):
        c = lax.axis_index("c")
        s = lax.axis_index("s")
        rb0 = c * RB_SC + s * RB_TILE
        nchunks = RB_TILE // 8

        nb = (c * NS + s) * NRB_TILE
        pltpu.sync_copy(tei0.at[pl.ds(nb, NRB_TILE)], idxn)
        ndescs = [pltpu.async_copy(ttv.at[idxn.at[j]], ntv.at[j], semn)
                  for j in range(NRB_TILE)]

        def zb(r, _):
            zbuf[r, :] = jnp.zeros((L,), _f32)
            return 0
        lax.fori_loop(0, 320, zb, 0)
        zdesc = pltpu.async_copy(zbuf, spm.at[pl.ds(s * 320, 320)], semz)

        pltpu.sync_copy(tei2d.at[pl.ds(rb0, RB_TILE)], idx_all_s)

        def dstrow(r, _):
            for v in range(8):
                ibase = (rb0 + r) * 128 + v * L
                ivec = lax.broadcasted_iota(_i32, (L,), 0) + ibase
                node = lax.shift_right_logical(ivec, 4)
                sub = jnp.where(ivec >= PAIR_SPLIT, jnp.int32(NODE_SPLIT),
                                jnp.int32(0))
                idx_all_d[r, pl.ds(v * L, L)] = node - sub
            return 0
        lax.fori_loop(0, RB_TILE, dstrow, 0)

        def fire_gathers(blk):
            par = blk % 2
            return [pltpu.async_copy(
                msg.at[idx_all_s.at[blk * 8 + j]],
                rows.at[pl.ds(par * 1024 + j * 128, 128)], semg)
                for j in range(8)]

        gdescs = fire_gathers(0)
        for dsc in ndescs:
            dsc.wait()
        pltpu.sync_copy(ntv, nt_out.at[pl.ds(nb, NRB_TILE)])
        zdesc.wait()
        plsc.subcore_barrier()

        sdescs_prev = None
        for blk in range(nchunks):
            par = blk % 2
            for dsc in gdescs:
                dsc.wait()
            if blk + 1 < nchunks:
                if sdescs_prev is not None:
                    for dsc in sdescs_prev:
                        dsc.wait()
                    sdescs_prev = None
                next_gdescs = fire_gathers(blk + 1)
            sdescs = [pltpu.async_copy(
                rows.at[pl.ds(par * 1024 + j * 128, 128)],
                spm.at[idx_all_d.at[blk * 8 + j]], sems, add=True)
                for j in range(8)]
            if blk + 1 < nchunks:
                sdescs_prev, gdescs = sdescs, next_gdescs
            else:
                for dsc in sdescs:
                    dsc.wait()
        if sdescs_prev is not None:
            for dsc in sdescs_prev:
                dsc.wait()
        plsc.subcore_barrier()

        @pl.when(c == 0)
        def _():
            base = s * 320
            pltpu.sync_copy(spm.at[pl.ds(base, 320)],
                            aggr_out.at[pl.ds(base, 320)])

        @pl.when(c == 1)
        def _():
            base = s * 305
            pltpu.sync_copy(spm.at[pl.ds(base, 305)],
                            aggr_out.at[pl.ds(NODE_SPLIT + base, 305)])

    return k7


BN8 = 2000


def _k8_body(aggr_r, nt_r, wae_r, bae_r, wxe_r, bxe_r, out_r):
    nt = nt_r[:, 0:1]
    oh = _onehot(nt, A, BN8)
    w = _softmax_lanes(
        jnp.dot(oh, wae_r[...], preferred_element_type=_f32) + bae_r[...])
    basis = _silu(
        jnp.dot(aggr_r[...], wxe_r[...], preferred_element_type=_f32) + bxe_r[...])
    pe = jnp.sum(w * basis)

    @pl.when(pl.program_id(0) == 0)
    def _():
        out_r[...] = jnp.reshape(pe, (1, 1))

    @pl.when(pl.program_id(0) > 0)
    def _():
        out_r[...] = out_r[...] + jnp.reshape(pe, (1, 1))


def _k8(aggr, nt2, p):
    full = lambda shape: pl.BlockSpec(shape, lambda i: (0, 0))
    return pl.pallas_call(
        _k8_body,
        grid=(N // BN8,),
        in_specs=[
            pl.BlockSpec((BN8, M), lambda i: (i, 0)),
            pl.BlockSpec((BN8, 1), lambda i: (i, 0)),
            full((A, A)), full((1, A)),
            full((M, A)), full((1, A)),
        ],
        out_specs=pl.BlockSpec((1, 1), lambda i: (0, 0)),
        out_shape=jax.ShapeDtypeStruct((1, 1), _f32),
    )(aggr, nt2,
      p['Wa'], p['ba'].reshape(1, A),
      p['Wx'], p['bx'].reshape(1, A))


def kernel(target_edge_index, edge_source_index, edge, source_type,
           target_type, params):
    tei = target_edge_index.astype(_i32)
    esi = edge_source_index.astype(_i32)
    st = source_type.astype(_i32)
    tt = target_type.astype(_i32)

    st2 = st.reshape(E, 1)
    tt2 = tt.reshape(E, 1)

    tei_flat = tei.reshape(-1)
    pad = jnp.zeros((P_PAD - N * CUT,), _i32)
    tei2d = jnp.concatenate([tei_flat, pad]).reshape(RB, 128)
    esi_pad = jnp.concatenate([esi, pad]).reshape(RB, 128)
    tei0 = jnp.concatenate(
        [tei[:, 0], jnp.zeros((N_PAD - N,), _i32)]).reshape(NRB, 128)

    ex = edge[:, 0].reshape(ER, 128)
    ey = edge[:, 1].reshape(ER, 128)
    ez = edge[:, 2].reshape(ER, 128)
    dlane, binlane = _k0(ex, ey, ez)
    bine = binlane.reshape(E)
    msg, drm, w1, w2 = _k1(dlane.reshape(E, 1), st2, tt2, params)

    for _ in range(2):
        grid = _make_k3()(tei2d, bine, msg)
        g = _k4(grid.reshape(N, NBINS * M), params['spin_fc_W'],
                params['spin_fc_b'])
        x = _make_k5()(esi_pad, g)[:E]
        msg = _k6(x, drm, msg, w1, w2, params['m1_emb'], params['m2_emb'])

    aggr, nt2d = _make_k7()(tei2d, msg, tei0, tt)
    nt = nt2d.reshape(N_PAD)[:N].reshape(N, 1)
    energy = _k8(aggr, nt, params['e_emb'])
    return energy.reshape(())


# speedup vs baseline: 1.0802x; 1.0005x over previous
import functools

import jax
import jax.numpy as jnp
from jax import lax
from jax.experimental import pallas as pl
from jax.experimental.pallas import tpu as pltpu
from jax.experimental.pallas import tpu_sc as plsc

E = 160000
N = 10000
CUT = 16
NBINS = 16
M = 16
D = 64
A = 8
ET = A * A + A
DELTA = 6.0
SIGMA = 0.5

NC, NS, L = 2, 16, 16

P_PAD = 163840
RB = P_PAD // 128
RB_SC = RB // NC
RB_TILE = RB_SC // NS
PAIR_SPLIT = P_PAD // 2
GRID_ROWS_SC = 81920
NODE_SPLIT = PAIR_SPLIT // CUT
AGG_ROWS_SC = 5120

N_PAD = 12288
NRB = N_PAD // 128
NRB_TILE = NRB // (NC * NS)

_f32 = jnp.float32
_i32 = jnp.int32


def _mesh():
    return plsc.VectorSubcoreMesh(
        core_axis_name="c", subcore_axis_name="s", num_cores=NC,
        num_subcores=NS)


_SC_PARAMS = pltpu.CompilerParams(use_tc_tiling_on_sc=False,
                                 skip_device_barrier=True)


def _sigmoid(x):
    return 1.0 / (1.0 + jnp.exp(-x))


def _silu(x):
    return x * _sigmoid(x)


def _softmax_lanes(x):
    m = jnp.max(x, axis=1, keepdims=True)
    e = jnp.exp(x - m)
    return e / jnp.sum(e, axis=1, keepdims=True)


def _onehot(idx_col, width, be):
    i = lax.broadcasted_iota(_i32, (be, width), 1)
    return (i == idx_col).astype(_f32)


def _expand_mat(b, out):
    r = lax.broadcasted_iota(_i32, (b, b * out), 0)
    c = lax.broadcasted_iota(_i32, (b, b * out), 1)
    return (c // out == r).astype(_f32)


def _fold_mat(b, out):
    r = lax.broadcasted_iota(_i32, (b * out, out), 0)
    c = lax.broadcasted_iota(_i32, (b * out, out), 1)
    return (r % out == c).astype(_f32)


def _emb_combine(w, basis, b, out):
    wrep = jnp.dot(w, _expand_mat(b, out), preferred_element_type=_f32)
    prod = basis * wrep
    width = b * out
    while width > out:
        width //= 2
        prod = prod[:, :width] + prod[:, width:2 * width]
    return prod


ER = E // 128


def _k0_body(ex_r, ey_r, ez_r, d_r, bin_r):
    x = ex_r[...]
    y = ey_r[...]
    z = ez_r[...]
    d = jnp.sqrt(x * x + y * y + z * z)
    d_r[...] = d
    inv = 1.0 / (d + 1e-12)
    ux, uy, uz = x * inv, y * inv, z * inv
    cq = 0.7071067811865476
    lat = ((uz <= cq).astype(_i32) + (uz <= 0.0).astype(_i32)
           + (uz <= -cq).astype(_i32))
    lon = (jnp.logical_not((ux < 0.0) & (uy < 0.0)).astype(_i32)
           + (uy >= 0.0).astype(_i32)
           + ((ux <= 0.0) & (uy >= 0.0)).astype(_i32))
    bin_r[...] = lat * 4 + lon


def _k0(ex, ey, ez):
    blk = lambda: pl.BlockSpec((ER, 128), lambda i: (0, 0))
    return pl.pallas_call(
        _k0_body,
        grid=(1,),
        in_specs=[blk(), blk(), blk()],
        out_specs=[blk(), blk()],
        out_shape=[
            jax.ShapeDtypeStruct((ER, 128), _f32),
            jax.ShapeDtypeStruct((ER, 128), _i32),
        ],
    )(ex, ey, ez)


BE1 = 1600


def _k1_body(d_r, st_r, tt_r, scale_r, off_r, wi_r, bi_r, wm_r, bm_r,
             wasi_r, wati_r, bai_r, was1_r, wat1_r, ba1_r,
             was2_r, wat2_r, ba2_r, wx_r, bx_r,
             msg_r, drm_r, w1_r, w2_r):
    st = st_r[:, 0:1]
    tt = tt_r[:, 0:1]
    oh_s = _onehot(st, A, BE1)
    oh_t = _onehot(tt, A, BE1)

    p1r = lax.broadcasted_iota(_i32, (A, ET), 0)
    p1c = lax.broadcasted_iota(_i32, (A, ET), 1)
    p_src = (p1c % A == p1r).astype(_f32)
    p_tgt = (p1c // A == p1r).astype(_f32)
    oh_et = (jnp.dot(oh_s, p_src, preferred_element_type=_f32)
             * jnp.dot(oh_t, p_tgt, preferred_element_type=_f32))

    ones64 = jnp.ones((1, D), _f32)
    scw = jnp.dot(oh_et, jnp.dot(scale_r[...], ones64,
                                 preferred_element_type=_f32),
                  preferred_element_type=_f32)
    ofw = jnp.dot(oh_et, jnp.dot(off_r[...], ones64,
                                 preferred_element_type=_f32),
                  preferred_element_type=_f32)

    def wtab(was_r, wat_r, ba_r):
        logits = (jnp.dot(p_src.T, was_r[...], preferred_element_type=_f32)
                  + jnp.dot(p_tgt.T, wat_r[...], preferred_element_type=_f32)
                  + ba_r[...])
        return _softmax_lanes(logits)

    wi = jnp.dot(oh_et, wtab(wasi_r, wati_r, bai_r),
                 preferred_element_type=_f32)
    w1_r[...] = jnp.dot(oh_et, wtab(was1_r, wat1_r, ba1_r),
                        preferred_element_type=_f32)
    w2_r[...] = jnp.dot(oh_et, wtab(was2_r, wat2_r, ba2_r),
                        preferred_element_type=_f32)

    dw = jnp.dot(d_r[...], jnp.ones((1, D), _f32),
                 preferred_element_type=_f32)
    d2w = dw * scw + ofw
    cent = (lax.broadcasted_iota(_i32, (BE1, D), 1).astype(_f32)
            * (DELTA / (D - 1)))
    diff = d2w - cent
    raw = jnp.exp(diff * diff * (-1.0 / (2.0 * SIGMA * SIGMA)))

    dri = jnp.dot(raw, wi_r[...], preferred_element_type=_f32) + bi_r[...]
    drm_r[...] = jnp.dot(raw, wm_r[...], preferred_element_type=_f32) + bm_r[...]

    basis = _silu(jnp.dot(dri, wx_r[...], preferred_element_type=_f32) + bx_r[...])
    wrep = jnp.dot(wi, _expand_mat(A, M), preferred_element_type=_f32)
    msg_r[...] = jnp.dot(basis * wrep, _fold_mat(A, M),
                         preferred_element_type=_f32)


def _k1(d2col, st2, tt2, p):
    nblk = E // BE1
    full = lambda shape: pl.BlockSpec(shape, lambda i: (0, 0))
    p1, p2 = p['m1_emb'], p['m2_emb']
    return pl.pallas_call(
        _k1_body,
        grid=(nblk,),
        in_specs=[
            pl.BlockSpec((BE1, 1), lambda i: (i, 0)),
            pl.BlockSpec((BE1, 1), lambda i: (i, 0)),
            pl.BlockSpec((BE1, 1), lambda i: (i, 0)),
            full((ET, 1)), full((ET, 1)),
            full((D, D)), full((1, D)), full((D, D)), full((1, D)),
            full((A, A)), full((A, A)), full((1, A)),
            full((A, A)), full((A, A)), full((1, A)),
            full((A, A)), full((A, A)), full((1, A)),
            full((D, A * M)), full((1, A * M)),
        ],
        out_specs=[
            pl.BlockSpec((BE1, M), lambda i: (i, 0)),
            pl.BlockSpec((BE1, D), lambda i: (i, 0)),
            pl.BlockSpec((BE1, A), lambda i: (i, 0)),
            pl.BlockSpec((BE1, A), lambda i: (i, 0)),
        ],
        out_shape=[
            jax.ShapeDtypeStruct((E, M), _f32),
            jax.ShapeDtypeStruct((E, D), _f32),
            jax.ShapeDtypeStruct((E, A), _f32),
            jax.ShapeDtypeStruct((E, A), _f32),
        ],
    )(d2col, st2, tt2,
      p['dist_scale'].reshape(ET, 1), p['dist_offset'].reshape(ET, 1),
      p['init_fc_W'], p['init_fc_b'].reshape(1, D),
      p['msg_fc_W'], p['msg_fc_b'].reshape(1, D),
      p['init_emb']['Wa'][:A], p['init_emb']['Wa'][A:],
      p['init_emb']['ba'].reshape(1, A),
      p1['Wa'][:A], p1['Wa'][A:], p1['ba'].reshape(1, A),
      p2['Wa'][:A], p2['Wa'][A:], p2['ba'].reshape(1, A),
      p['init_emb']['Wx'], p['init_emb']['bx'].reshape(1, A * M))


@functools.lru_cache(maxsize=None)
def _make_k3():
    @functools.partial(
        pl.kernel, mesh=_mesh(), compiler_params=_SC_PARAMS,
        out_type=jax.ShapeDtypeStruct((E, M), _f32),
        scratch_types=[
            pltpu.VMEM((RB_TILE, 128), _i32),
            pltpu.VMEM((RB_TILE, 128), _i32),
            pltpu.VMEM((2048, M), _f32),
            pltpu.VMEM((256, M), _f32),
            pltpu.VMEM_SHARED((GRID_ROWS_SC, M), _f32),
            pltpu.SemaphoreType.DMA,
            pltpu.SemaphoreType.DMA,
            pltpu.SemaphoreType.DMA,
            pltpu.SemaphoreType.DMA,
        ],
    )
    def k3(tei2d, bine, msg, grid_out, idx_all_s, idx_all_d, rows,
           zbuf, spm, semg, sems, semz, semb):
        c = lax.axis_index("c")
        s = lax.axis_index("s")
        rb0 = c * RB_SC + s * RB_TILE
        nchunks = RB_TILE // 8

        def zb(r, _):
            zbuf[r, :] = jnp.zeros((L,), _f32)
            return 0
        lax.fori_loop(0, 256, zb, 0)

        rows_tile = GRID_ROWS_SC // NS
        zdescs = [pltpu.async_copy(
            zbuf, spm.at[pl.ds(s * rows_tile + q * 256, 256)], semz)
            for q in range(rows_tile // 256)]

        pltpu.sync_copy(tei2d.at[pl.ds(rb0, RB_TILE)], idx_all_s)

        for grp in range(RB_TILE // 8):
            bdescs = [pltpu.async_copy(
                bine.at[idx_all_s.at[grp * 8 + j]],
                idx_all_d.at[grp * 8 + j], semb) for j in range(8)]
            for dsc in bdescs:
                dsc.wait()

        def dstrow(r, _):
            for v in range(8):
                ibase = (rb0 + r) * 128 + v * L
                ivec = lax.broadcasted_iota(_i32, (L,), 0) + ibase
                row16 = ivec & jnp.int32(-16)
                sub = jnp.where(ivec >= PAIR_SPLIT, jnp.int32(PAIR_SPLIT),
                                jnp.int32(0))
                bvec = idx_all_d[r, pl.ds(v * L, L)]
                idx_all_d[r, pl.ds(v * L, L)] = row16 - sub + bvec
            return 0
        lax.fori_loop(0, RB_TILE, dstrow, 0)

        def fire_gathers(blk):
            par = blk % 2
            return [pltpu.async_copy(
                msg.at[idx_all_s.at[blk * 8 + j]],
                rows.at[pl.ds(par * 1024 + j * 128, 128)], semg)
                for j in range(8)]

        gdescs = fire_gathers(0)
        for dsc in zdescs:
            dsc.wait()
        plsc.subcore_barrier()

        sdescs_prev = None
        for blk in range(nchunks):
            par = blk % 2
            for dsc in gdescs:
                dsc.wait()
            if blk + 1 < nchunks:
                if sdescs_prev is not None:
                    for dsc in sdescs_prev:
                        dsc.wait()
                    sdescs_prev = None
                next_gdescs = fire_gathers(blk + 1)
            sdescs = [pltpu.async_copy(
                rows.at[pl.ds(par * 1024 + j * 128, 128)],
                spm.at[idx_all_d.at[blk * 8 + j]], sems, add=True)
                for j in range(8)]
            if blk + 1 < nchunks:
                sdescs_prev, gdescs = sdescs, next_gdescs
            else:
                for dsc in sdescs:
                    dsc.wait()
        if sdescs_prev is not None:
            for dsc in sdescs_prev:
                dsc.wait()
        plsc.subcore_barrier()

        @pl.when(c == 0)
        def _():
            base = s * 5120
            pltpu.sync_copy(spm.at[pl.ds(base, 5120)],
                            grid_out.at[pl.ds(base, 5120)])

        @pl.when(c == 1)
        def _():
            base = s * 4880
            pltpu.sync_copy(spm.at[pl.ds(base, 4880)],
                            grid_out.at[pl.ds(PAIR_SPLIT + base, 4880)])

    return k3


BN4 = 2000


def _k4_body(grid_r, w_r, b_r, g_r):
    g_r[...] = jnp.dot(grid_r[...], w_r[...], preferred_element_type=_f32) + b_r[...]


def _k4(grid, w, b):
    return pl.pallas_call(
        _k4_body,
        grid=(N // BN4,),
        in_specs=[
            pl.BlockSpec((BN4, NBINS * M), lambda i: (i, 0)),
            pl.BlockSpec((NBINS * M, D), lambda i: (0, 0)),
            pl.BlockSpec((1, D), lambda i: (0, 0)),
        ],
        out_specs=pl.BlockSpec((BN4, D), lambda i: (i, 0)),
        out_shape=jax.ShapeDtypeStruct((N, D), _f32),
    )(grid, w, b.reshape(1, D))


K5_RB_SC0 = 896
K5_RB_SC1 = RB - K5_RB_SC0


@functools.lru_cache(maxsize=None)
def _make_k5():
    @functools.partial(
        pl.kernel, mesh=_mesh(), compiler_params=_SC_PARAMS,
        out_type=jax.ShapeDtypeStruct((P_PAD, D), _f32),
        scratch_types=[
            pltpu.VMEM((K5_RB_SC0 // NS, 128), _i32),
            pltpu.VMEM((1024, D), _f32),
            pltpu.SemaphoreType.DMA,
            pltpu.SemaphoreType.DMA,
        ],
    )
    def k5(esi2d, g, x_out, idx_all, rows, semg, semw):
        c = lax.axis_index("c")
        s = lax.axis_index("s")

        def pipeline(rb0, rb_tile):
            nchunks = rb_tile // 4
            pltpu.sync_copy(esi2d.at[pl.ds(rb0, rb_tile)],
                            idx_all.at[pl.ds(0, rb_tile)])

            def fire_gathers(blk):
                par = blk % 2
                return [pltpu.async_copy(
                    g.at[idx_all.at[blk * 4 + j]],
                    rows.at[pl.ds(par * 512 + j * 128, 128)], semg)
                    for j in range(4)]

            gdescs = fire_gathers(0)
            wdesc_prev = None
            for blk in range(nchunks):
                par = blk % 2
                for dsc in gdescs:
                    dsc.wait()
                if blk + 1 < nchunks:
                    if wdesc_prev is not None:
                        wdesc_prev.wait()
                        wdesc_prev = None
                    next_gdescs = fire_gathers(blk + 1)
                wdesc = pltpu.async_copy(
                    rows.at[pl.ds(par * 512, 512)],
                    x_out.at[pl.ds((rb0 + blk * 4) * 128, 512)], semw)
                if blk + 1 < nchunks:
                    wdesc_prev, gdescs = wdesc, next_gdescs
                else:
                    wdesc.wait()
            if wdesc_prev is not None:
                wdesc_prev.wait()

        @pl.when(c == 0)
        def _():
            pipeline(s * (K5_RB_SC0 // NS), K5_RB_SC0 // NS)

        @pl.when(c == 1)
        def _():
            pipeline(K5_RB_SC0 + s * (K5_RB_SC1 // NS), K5_RB_SC1 // NS)

    return k5


BE6 = 1600


def _k6_body(x_r, drm_r, msg_r, w1_r, w2_r,
             wx1_r, bx1_r, wx2s_r, bx2_r,
             out_r):
    w1 = w1_r[...]
    w2 = w2_r[...]
    b1 = _silu(jnp.dot(x_r[...], wx1_r[...], preferred_element_type=_f32)
               + bx1_r[...])
    wrep1 = jnp.dot(w1, _expand_mat(A, D), preferred_element_type=_f32)
    prod1 = b1 * wrep1
    f1 = prod1[:, :256] + prod1[:, 256:]
    f2 = f1[:, :128] + f1[:, 128:]
    drm_pad = jnp.concatenate(
        [drm_r[...], jnp.zeros((BE6, D), _f32)], axis=1)
    t2 = f2 + drm_pad
    b2 = _silu(jnp.dot(t2, wx2s_r[...], preferred_element_type=_f32)
               + bx2_r[...])
    wrep2 = jnp.dot(w2, _expand_mat(A, M), preferred_element_type=_f32)
    res = jnp.dot(b2 * wrep2, _fold_mat(A, M), preferred_element_type=_f32)
    out_r[...] = msg_r[...] + res


def _k6(x, drm, msg, w1, w2, p1, p2):
    nblk = E // BE6
    full = lambda shape: pl.BlockSpec(shape, lambda i: (0, 0))
    wx2_stacked = jnp.concatenate([p2['Wx'], p2['Wx']], axis=0)
    return pl.pallas_call(
        _k6_body,
        grid=(nblk,),
        in_specs=[
            pl.BlockSpec((BE6, D), lambda i: (i, 0)),
            pl.BlockSpec((BE6, D), lambda i: (i, 0)),
            pl.BlockSpec((BE6, M), lambda i: (i, 0)),
            pl.BlockSpec((BE6, A), lambda i: (i, 0)),
            pl.BlockSpec((BE6, A), lambda i: (i, 0)),
            full((D, A * D)), full((1, A * D)),
            full((2 * D, A * M)), full((1, A * M)),
        ],
        out_specs=pl.BlockSpec((BE6, M), lambda i: (i, 0)),
        out_shape=jax.ShapeDtypeStruct((E, M), _f32),
    )(x, drm, msg, w1, w2,
      p1['Wx'], p1['bx'].reshape(1, A * D),
      wx2_stacked, p2['bx'].reshape(1, A * M))


@functools.lru_cache(maxsize=None)
def _make_k7():
    @functools.partial(
        pl.kernel, mesh=_mesh(), compiler_params=_SC_PARAMS,
        out_type=[jax.ShapeDtypeStruct((N, M), _f32),
                  jax.ShapeDtypeStruct((NRB, 128), _i32)],
        scratch_types=[
            pltpu.VMEM((RB_TILE, 128), _i32),
            pltpu.VMEM((RB_TILE, 128), _i32),
            pltpu.VMEM((2048, M), _f32),
            pltpu.VMEM((320, M), _f32),
            pltpu.VMEM((NRB_TILE, 128), _i32),
            pltpu.VMEM((NRB_TILE, 128), _i32),
            pltpu.VMEM_SHARED((AGG_ROWS_SC, M), _f32),
            pltpu.SemaphoreType.DMA,
            pltpu.SemaphoreType.DMA,
            pltpu.SemaphoreType.DMA,
            pltpu.SemaphoreType.DMA,
        ],
    )
    def k7(tei2d, msg, tei0, ttv, aggr_out, nt_out,
           idx_all_s, idx_all_d, rows, zbuf, idxn, ntv, spm,
           semg, sems, semz, semn):
        c = lax.axis_index("c")
        s = lax.axis_index("s")
        rb0 = c * RB_SC + s * RB_TILE
        nchunks = RB_TILE // 8

        nb = (c * NS + s) * NRB_TILE
        pltpu.sync_copy(tei0.at[pl.ds(nb, NRB_TILE)], idxn)
        ndescs = [pltpu.async_copy(ttv.at[idxn.at[j]], ntv.at[j], semn)
                  for j in range(NRB_TILE)]

        def zb(r, _):
            zbuf[r, :] = jnp.zeros((L,), _f32)
            return 0
        lax.fori_loop(0, 320, zb, 0)
        zdesc = pltpu.async_copy(zbuf, spm.at[pl.ds(s * 320, 320)], semz)

        pltpu.sync_copy(tei2d.at[pl.ds(rb0, RB_TILE)], idx_all_s)

        def dstrow(r, _):
            for v in range(8):
                ibase = (rb0 + r) * 128 + v * L
                ivec = lax.broadcasted_iota(_i32, (L,), 0) + ibase
                node = lax.shift_right_logical(ivec, 4)
                sub = jnp.where(ivec >= PAIR_SPLIT, jnp.int32(NODE_SPLIT),
                                jnp.int32(0))
                idx_all_d[r, pl.ds(v * L, L)] = node - sub
            return 0
        lax.fori_loop(0, RB_TILE, dstrow, 0)

        def fire_gathers(blk):
            par = blk % 2
            return [pltpu.async_copy(
                msg.at[idx_all_s.at[blk * 8 + j]],
                rows.at[pl.ds(par * 1024 + j * 128, 128)], semg)
                for j in range(8)]

        gdescs = fire_gathers(0)
        for dsc in ndescs:
            dsc.wait()
        pltpu.sync_copy(ntv, nt_out.at[pl.ds(nb, NRB_TILE)])
        zdesc.wait()
        plsc.subcore_barrier()

        sdescs_prev = None
        for blk in range(nchunks):
            par = blk % 2
            for dsc in gdescs:
                dsc.wait()
            if blk + 1 < nchunks:
                if sdescs_prev is not None:
                    for dsc in sdescs_prev:
                        dsc.wait()
                    sdescs_prev = None
                next_gdescs = fire_gathers(blk + 1)
            sdescs = [pltpu.async_copy(
                rows.at[pl.ds(par * 1024 + j * 128, 128)],
                spm.at[idx_all_d.at[blk * 8 + j]], sems, add=True)
                for j in range(8)]
            if blk + 1 < nchunks:
                sdescs_prev, gdescs = sdescs, next_gdescs
            else:
                for dsc in sdescs:
                    dsc.wait()
        if sdescs_prev is not None:
            for dsc in sdescs_prev:
                dsc.wait()
        plsc.subcore_barrier()

        @pl.when(c == 0)
        def _():
            base = s * 320
            pltpu.sync_copy(spm.at[pl.ds(base, 320)],
                            aggr_out.at[pl.ds(base, 320)])

        @pl.when(c == 1)
        def _():
            base = s * 305
            pltpu.sync_copy(spm.at[pl.ds(base, 305)],
                            aggr_out.at[pl.ds(NODE_SPLIT + base, 305)])

    return k7


BN8 = 2000


def _k8_body(aggr_r, nt_r, wae_r, bae_r, wxe_r, bxe_r, out_r):
    nt = nt_r[:, 0:1]
    oh = _onehot(nt, A, BN8)
    w = _softmax_lanes(
        jnp.dot(oh, wae_r[...], preferred_element_type=_f32) + bae_r[...])
    basis = _silu(
        jnp.dot(aggr_r[...], wxe_r[...], preferred_element_type=_f32) + bxe_r[...])
    pe = jnp.sum(w * basis)

    @pl.when(pl.program_id(0) == 0)
    def _():
        out_r[...] = jnp.reshape(pe, (1, 1))

    @pl.when(pl.program_id(0) > 0)
    def _():
        out_r[...] = out_r[...] + jnp.reshape(pe, (1, 1))


def _k8(aggr, nt2, p):
    full = lambda shape: pl.BlockSpec(shape, lambda i: (0, 0))
    return pl.pallas_call(
        _k8_body,
        grid=(N // BN8,),
        in_specs=[
            pl.BlockSpec((BN8, M), lambda i: (i, 0)),
            pl.BlockSpec((BN8, 1), lambda i: (i, 0)),
            full((A, A)), full((1, A)),
            full((M, A)), full((1, A)),
        ],
        out_specs=pl.BlockSpec((1, 1), lambda i: (0, 0)),
        out_shape=jax.ShapeDtypeStruct((1, 1), _f32),
    )(aggr, nt2,
      p['Wa'], p['ba'].reshape(1, A),
      p['Wx'], p['bx'].reshape(1, A))


def kernel(target_edge_index, edge_source_index, edge, source_type,
           target_type, params):
    tei = target_edge_index.astype(_i32)
    esi = edge_source_index.astype(_i32)
    st = source_type.astype(_i32)
    tt = target_type.astype(_i32)

    st2 = st.reshape(E, 1)
    tt2 = tt.reshape(E, 1)

    tei_flat = tei.reshape(-1)
    pad = jnp.zeros((P_PAD - N * CUT,), _i32)
    tei2d = jnp.concatenate([tei_flat, pad]).reshape(RB, 128)
    esi_pad = jnp.concatenate([esi, pad]).reshape(RB, 128)
    tei0 = jnp.concatenate(
        [tei[:, 0], jnp.zeros((N_PAD - N,), _i32)]).reshape(NRB, 128)

    ex = edge[:, 0].reshape(ER, 128)
    ey = edge[:, 1].reshape(ER, 128)
    ez = edge[:, 2].reshape(ER, 128)
    dlane, binlane = _k0(ex, ey, ez)
    bine = binlane.reshape(E)
    msg, drm, w1, w2 = _k1(dlane.reshape(E, 1), st2, tt2, params)

    for _ in range(2):
        grid = _make_k3()(tei2d, bine, msg)
        g = _k4(grid.reshape(N, NBINS * M), params['spin_fc_W'],
                params['spin_fc_b'])
        x = _make_k5()(esi_pad, g)[:E]
        msg = _k6(x, drm, msg, w1, w2, params['m1_emb'], params['m2_emb'])

    aggr, nt2d = _make_k7()(tei2d, msg, tei0, tt)
    nt = nt2d.reshape(N_PAD)[:N].reshape(N, 1)
    energy = _k8(aggr, nt, params['e_emb'])
    return energy.reshape(())
